# Initial kernel scaffold; baseline (speedup 1.0000x reference)
import jax
import jax.numpy as jnp
from jax import lax
from jax.experimental import pallas as pl
from jax.experimental.pallas import tpu as pltpu
from jax.experimental.pallas import tpu_sc as plsc

L = 16

B, C, H, W = 4, 3, 384, 384
M = 32
NPLANE = B * C
P = H * W
NW = 32
PPW = P // NW
K = 512
CPW = PPW // K


def _body(x_hbm, emb_hbm, out_hbm, emb_v, bt_v, x_v, out_v):
    nc = 2
    wid = lax.axis_index("s") * nc + lax.axis_index("c")

    pltpu.sync_copy(emb_hbm, emb_v)

    lane = lax.iota(jnp.int32, L)

    for m in range(M):
        def build_g(g, _, m=m):
            vidx = (g * L + lane) * M + m
            e = plsc.load_gather(emb_v, [vidx])
            bt_v[pl.ds(m * 256 + g * L, L)] = (jnp.sign(e) + 1.0) * 0.5
            return 0

        lax.fori_loop(0, 256 // L, build_g, 0)

    def do_chunk(plane, j):
        base = wid * PPW + j * K
        pltpu.sync_copy(x_hbm.at[pl.ds(plane * P + base, K)], x_v)

        def do_group(g, _):
            x16 = x_v[pl.ds(g * L, L)]
            idx = jnp.clip((x16 * 255.0).astype(jnp.int32), 0, 255)
            for m in range(M):
                out_v[m, pl.ds(g * L, L)] = plsc.load_gather(
                    bt_v, [idx + (m * 256)])
            return 0

        lax.fori_loop(0, K // L, do_group, 0)
        pltpu.sync_copy(out_v, out_hbm.at[plane, :, pl.ds(base, K)])

    for plane in range(NPLANE):
        def plane_loop(j, _, plane=plane):
            do_chunk(plane, j)
            return 0

        lax.fori_loop(0, CPW, plane_loop, 0)


@jax.jit
def kernel(x, embedding):
    x_flat = x.reshape(-1)
    emb_flat = embedding.reshape(-1)
    mesh = plsc.VectorSubcoreMesh(core_axis_name="c", subcore_axis_name="s")
    out = pl.kernel(
        _body,
        out_type=jax.ShapeDtypeStruct((NPLANE, M, P), jnp.float32),
        mesh=mesh,
        scratch_types=[
            pltpu.VMEM((256 * M,), jnp.float32),
            pltpu.VMEM((M * 256,), jnp.float32),
            pltpu.VMEM((K,), jnp.float32),
            pltpu.VMEM((M, K), jnp.float32),
        ],
    )(x_flat, emb_flat)
    return out.reshape(B, C * M, H, W)


# speedup vs baseline: 8.9129x; 8.9129x over previous
import jax
import jax.numpy as jnp
from jax import lax
from jax.experimental import pallas as pl
from jax.experimental.pallas import tpu as pltpu
from jax.experimental.pallas import tpu_sc as plsc

L = 16

B, C, H, W = 4, 3, 384, 384
M = 32
NPLANE = B * C
P = H * W
NW = 32
PPW = P // NW
K = 512
CPW = PPW // K


def _body(x_hbm, emb_hbm, out_hbm, emb_v, bt_v, x_v, out_v):
    nc = 2
    wid = lax.axis_index("s") * nc + lax.axis_index("c")

    pltpu.sync_copy(emb_hbm, emb_v)

    lane = lax.iota(jnp.int32, L)

    for m in range(M):
        def build_g(g, _, m=m):
            vidx = (g * L + lane) * M + m
            e = plsc.load_gather(emb_v, [vidx])
            bt_v[pl.ds(m * 256 + g * L, L)] = (jnp.sign(e) + 1.0) * 0.5
            return 0

        lax.fori_loop(0, 256 // L, build_g, 0)

    def do_chunk(plane, j):
        base = wid * PPW + j * K
        pltpu.sync_copy(x_hbm.at[pl.ds(plane * P + base, K)], x_v)

        def do_group(g, _):
            x16 = x_v[pl.ds(g * L, L)]
            idx = jnp.clip((x16 * 255.0).astype(jnp.int32), 0, 255)
            for m in range(M):
                out_v[m, pl.ds(g * L, L)] = plsc.load_gather(
                    bt_v, [idx + (m * 256)])
            return 0

        lax.fori_loop(0, K // L, do_group, 0)
        pltpu.sync_copy(out_v, out_hbm.at[plane, :, pl.ds(base, K)])

    for plane in range(NPLANE):
        def plane_loop(j, _, plane=plane):
            do_chunk(plane, j)
            return 0

        lax.fori_loop(0, CPW, plane_loop, 0)


@jax.jit
def kernel(x, embedding):
    x_flat = x.reshape(-1)
    emb_flat = embedding.reshape(-1)
    mesh = plsc.VectorSubcoreMesh(core_axis_name="c", subcore_axis_name="s")
    out = pl.kernel(
        _body,
        out_type=jax.ShapeDtypeStruct((NPLANE, M, P), jnp.float32),
        mesh=mesh,
        compiler_params=pltpu.CompilerParams(needs_layout_passes=False),
        scratch_types=[
            pltpu.VMEM((256 * M,), jnp.float32),
            pltpu.VMEM((M * 256,), jnp.float32),
            pltpu.VMEM((K,), jnp.float32),
            pltpu.VMEM((M, K), jnp.float32),
        ],
    )(x_flat, emb_flat)
    return out.reshape(B, C * M, H, W)


# speedup vs baseline: 10.5885x; 1.1880x over previous
import jax
import jax.numpy as jnp
from jax import lax
from jax.experimental import pallas as pl
from jax.experimental.pallas import tpu as pltpu
from jax.experimental.pallas import tpu_sc as plsc

L = 16

B, C, H, W = 4, 3, 384, 384
M = 32
NPLANE = B * C
P = H * W
NW = 32
PPW = P // NW
CPW = 6
K = PPW // CPW


def _body(x_hbm, emb_hbm, out_hbm, emb_v, bt_v, x_v, out_v,
          xs0, xs1, os0, os1):
    nc = 2
    wid = lax.axis_index("s") * nc + lax.axis_index("c")
    xsems = (xs0, xs1)
    osems = (os0, os1)
    slab = wid * PPW

    pltpu.sync_copy(emb_hbm, emb_v)

    lane = lax.iota(jnp.int32, L)

    for m in range(M):
        def build_g(g, _, m=m):
            vidx = (g * L + lane) * M + m
            e = plsc.load_gather(emb_v, [vidx])
            bt_v[pl.ds(m * 256 + g * L, L)] = (jnp.sign(e) + 1.0) * 0.5
            return 0

        lax.fori_loop(0, 256 // L, build_g, 0)

    for bb in range(2):
        pltpu.async_copy(
            x_hbm.at[pl.ds(slab + bb * K, K)], x_v.at[bb], xsems[bb])

    def plane_step(plane, _):
        for j in range(CPW):
            bb = j % 2
            base = slab + j * K

            pltpu.make_async_copy(
                x_hbm.at[pl.ds(0, K)], x_v.at[bb], xsems[bb]).wait()

            if j >= 2:
                pltpu.make_async_copy(
                    out_v.at[bb], out_hbm.at[0, :, pl.ds(0, K)],
                    osems[bb]).wait()
            else:
                @pl.when(plane > 0)
                def _wait_out(bb=bb):
                    pltpu.make_async_copy(
                        out_v.at[bb], out_hbm.at[0, :, pl.ds(0, K)],
                        osems[bb]).wait()

            def do_group(g, _, bb=bb):
                x16 = x_v[bb, pl.ds(g * L, L)]
                idx = jnp.clip((x16 * 255.0).astype(jnp.int32), 0, 255)
                for m in range(M):
                    out_v[bb, m, pl.ds(g * L, L)] = plsc.load_gather(
                        bt_v, [idx + (m * 256)])
                return 0

            lax.fori_loop(0, K // L, do_group, 0, unroll=2)

            pltpu.async_copy(
                out_v.at[bb], out_hbm.at[plane, :, pl.ds(base, K)],
                osems[bb])

            if j + 2 < CPW:
                pltpu.async_copy(
                    x_hbm.at[pl.ds(plane * P + slab + (j + 2) * K, K)],
                    x_v.at[bb], xsems[bb])
            else:
                @pl.when(plane + 1 < NPLANE)
                def _prefetch(bb=bb, j=j):
                    pltpu.async_copy(
                        x_hbm.at[pl.ds((plane + 1) * P + slab
                                       + (j + 2 - CPW) * K, K)],
                        x_v.at[bb], xsems[bb])
        return 0

    lax.fori_loop(0, NPLANE, plane_step, 0)

    for bb in range(2):
        pltpu.make_async_copy(
            out_v.at[bb], out_hbm.at[0, :, pl.ds(0, K)], osems[bb]).wait()


@jax.jit
def kernel(x, embedding):
    x_flat = x.reshape(-1)
    emb_flat = embedding.reshape(-1)
    mesh = plsc.VectorSubcoreMesh(core_axis_name="c", subcore_axis_name="s")
    out = pl.kernel(
        _body,
        out_type=jax.ShapeDtypeStruct((NPLANE, M, P), jnp.float32),
        mesh=mesh,
        compiler_params=pltpu.CompilerParams(needs_layout_passes=False),
        scratch_types=[
            pltpu.VMEM((256 * M,), jnp.float32),
            pltpu.VMEM((M * 256,), jnp.float32),
            pltpu.VMEM((2, K), jnp.float32),
            pltpu.VMEM((2, M, K), jnp.float32),
            pltpu.SemaphoreType.DMA,
            pltpu.SemaphoreType.DMA,
            pltpu.SemaphoreType.DMA,
            pltpu.SemaphoreType.DMA,
        ],
    )(x_flat, emb_flat)
    return out.reshape(B, C * M, H, W)


# speedup vs baseline: 16.7333x; 1.5803x over previous
import jax
import jax.numpy as jnp
from jax import lax
from jax.experimental import pallas as pl
from jax.experimental.pallas import tpu as pltpu
from jax.experimental.pallas import tpu_sc as plsc

L = 16

B, C, H, W = 4, 3, 384, 384
M = 32
NPLANE = B * C
P = H * W
NW = 32
PPW = P // NW
CPW = 6
K = PPW // CPW


def _body(x_hbm, emb_hbm, out_hbm, emb_v, bt_v, x_v, out_v,
          xs0, xs1, os0, os1):
    nc = 2
    wid = lax.axis_index("s") * nc + lax.axis_index("c")
    xsems = (xs0, xs1)
    osems = (os0, os1)
    slab = wid * PPW

    pltpu.sync_copy(emb_hbm, emb_v)

    lane = lax.iota(jnp.int32, L)

    for m in range(M):
        def build_g(g, _, m=m):
            vidx = (g * L + lane) * M + m
            e = plsc.load_gather(emb_v, [vidx])
            bt_v[pl.ds(m * 256 + g * L, L)] = (jnp.sign(e) + 1.0) * 0.5
            return 0

        lax.fori_loop(0, 256 // L, build_g, 0)

    for bb in range(2):
        pltpu.async_copy(
            x_hbm.at[pl.ds(slab + bb * K, K)], x_v.at[bb], xsems[bb])

    def plane_step(plane, _):
        for j in range(CPW):
            bb = j % 2
            base = slab + j * K

            pltpu.make_async_copy(
                x_hbm.at[pl.ds(0, K)], x_v.at[bb], xsems[bb]).wait()

            if j >= 2:
                pltpu.make_async_copy(
                    out_v.at[bb], out_hbm.at[0, :, pl.ds(0, K)],
                    osems[bb]).wait()
            else:
                @pl.when(plane > 0)
                def _wait_out(bb=bb):
                    pltpu.make_async_copy(
                        out_v.at[bb], out_hbm.at[0, :, pl.ds(0, K)],
                        osems[bb]).wait()

            @plsc.parallel_loop(0, K // L, unroll=2)
            def do_group(g, bb=bb):
                x16 = x_v[bb, pl.ds(g * L, L)]
                idx = jnp.clip((x16 * 255.0).astype(jnp.int32), 0, 255)
                vals = [plsc.load_gather(bt_v, [idx + (m * 256)])
                        for m in range(M)]
                for m in range(M):
                    out_v[bb, m, pl.ds(g * L, L)] = vals[m]

            pltpu.async_copy(
                out_v.at[bb], out_hbm.at[plane, :, pl.ds(base, K)],
                osems[bb])

            if j + 2 < CPW:
                pltpu.async_copy(
                    x_hbm.at[pl.ds(plane * P + slab + (j + 2) * K, K)],
                    x_v.at[bb], xsems[bb])
            else:
                @pl.when(plane + 1 < NPLANE)
                def _prefetch(bb=bb, j=j):
                    pltpu.async_copy(
                        x_hbm.at[pl.ds((plane + 1) * P + slab
                                       + (j + 2 - CPW) * K, K)],
                        x_v.at[bb], xsems[bb])
        return 0

    lax.fori_loop(0, NPLANE, plane_step, 0)

    for bb in range(2):
        pltpu.make_async_copy(
            out_v.at[bb], out_hbm.at[0, :, pl.ds(0, K)], osems[bb]).wait()


@jax.jit
def kernel(x, embedding):
    x_flat = x.reshape(-1)
    emb_flat = embedding.reshape(-1)
    mesh = plsc.VectorSubcoreMesh(core_axis_name="c", subcore_axis_name="s")
    out = pl.kernel(
        _body,
        out_type=jax.ShapeDtypeStruct((NPLANE, M, P), jnp.float32),
        mesh=mesh,
        compiler_params=pltpu.CompilerParams(needs_layout_passes=False),
        scratch_types=[
            pltpu.VMEM((256 * M,), jnp.float32),
            pltpu.VMEM((M * 256,), jnp.float32),
            pltpu.VMEM((2, K), jnp.float32),
            pltpu.VMEM((2, M, K), jnp.float32),
            pltpu.SemaphoreType.DMA,
            pltpu.SemaphoreType.DMA,
            pltpu.SemaphoreType.DMA,
            pltpu.SemaphoreType.DMA,
        ],
    )(x_flat, emb_flat)
    return out.reshape(B, C * M, H, W)


# speedup vs baseline: 35.6954x; 2.1332x over previous
import jax
import jax.numpy as jnp
from jax import lax
from jax.experimental import pallas as pl
from jax.experimental.pallas import tpu as pltpu
from jax.experimental.pallas import tpu_sc as plsc

L = 16

B, C, H, W = 4, 3, 384, 384
M = 32
MH = M // 2
NPLANE = B * C
NW = 32
RS = 8
SPP = H // RS
NST = NPLANE * SPP
SPW = NST // NW


def _adv(p, r):
    r2 = r + 1
    wrap = r2 >= SPP
    return jnp.where(wrap, p + 1, p), jnp.where(wrap, 0, r2)


def _body(x_hbm, emb_hbm, out_hbm, emb_v, bt_v, x_v, out_v,
          xs0, xs1, os0, os1):
    nc = 2
    wid = lax.axis_index("s") * nc + lax.axis_index("c")
    xsems = (xs0, xs1)
    osems = (os0, os1)

    pltpu.sync_copy(emb_hbm, emb_v)

    lane = lax.iota(jnp.int32, L)

    for m in range(M):
        def build_g(g, _, m=m):
            vidx = (g * L + lane) * M + m
            e = plsc.load_gather(emb_v, [vidx])
            bt_v[pl.ds(m * 256 + g * L, L)] = (jnp.sign(e) + 1.0) * 0.5
            return 0

        lax.fori_loop(0, 256 // L, build_g, 0)

    start = wid * SPW
    p0 = (start * 87382) >> 22
    r0 = start - p0 * SPP

    pa, ra = p0, r0
    for bb in range(2):
        pltpu.async_copy(
            x_hbm.at[pa, pl.ds(ra * RS, RS), :], x_v.at[bb], xsems[bb])
        pa, ra = _adv(pa, ra)

    def step(t, carry):
        p, r = carry
        for bb in range(2):
            pltpu.make_async_copy(
                x_hbm.at[0, pl.ds(0, RS), :], x_v.at[bb], xsems[bb]).wait()

            for hf in range(2):
                def wait_out(hf=hf):
                    pltpu.make_async_copy(
                        out_v.at[hf],
                        out_hbm.at[0, pl.ds(0, MH), pl.ds(0, RS), :],
                        osems[hf]).wait()

                if bb == 0:
                    @pl.when(t > 0)
                    def _wait0():
                        wait_out()
                else:
                    wait_out()

                for row in range(RS):
                    @plsc.parallel_loop(0, W // L, unroll=2)
                    def grp(g, bb=bb, hf=hf, row=row):
                        x16 = x_v[bb, row, pl.ds(g * L, L)]
                        idx = jnp.clip(
                            (x16 * 255.0).astype(jnp.int32), 0, 255)
                        vals = [
                            plsc.load_gather(
                                bt_v, [idx + ((hf * MH + mm) * 256)])
                            for mm in range(MH)]
                        for mm in range(MH):
                            out_v[hf, mm, row, pl.ds(g * L, L)] = vals[mm]

                pltpu.async_copy(
                    out_v.at[hf],
                    out_hbm.at[p, pl.ds(hf * MH, MH), pl.ds(r * RS, RS), :],
                    osems[hf])

            pn, rn = _adv(p, r)
            p2, r2 = _adv(pn, rn)
            s_next = t * 2 + bb + 2
            @pl.when(s_next < SPW)
            def _prefetch(bb=bb, p2=p2, r2=r2):
                pltpu.async_copy(
                    x_hbm.at[p2, pl.ds(r2 * RS, RS), :], x_v.at[bb],
                    xsems[bb])

            p, r = pn, rn
        return p, r

    lax.fori_loop(0, SPW // 2, step, (p0, r0))

    for hf in range(2):
        pltpu.make_async_copy(
            out_v.at[hf], out_hbm.at[0, pl.ds(0, MH), pl.ds(0, RS), :],
            osems[hf]).wait()


@jax.jit
def kernel(x, embedding):
    x3 = x.reshape(NPLANE, H, W)
    emb_flat = embedding.reshape(-1)
    mesh = plsc.VectorSubcoreMesh(core_axis_name="c", subcore_axis_name="s")
    out = pl.kernel(
        _body,
        out_type=jax.ShapeDtypeStruct((NPLANE, M, H, W), jnp.float32),
        mesh=mesh,
        compiler_params=pltpu.CompilerParams(needs_layout_passes=False),
        scratch_types=[
            pltpu.VMEM((256 * M,), jnp.float32),
            pltpu.VMEM((M * 256,), jnp.float32),
            pltpu.VMEM((2, RS, W), jnp.float32),
            pltpu.VMEM((2, MH, RS, W), jnp.float32),
            pltpu.SemaphoreType.DMA,
            pltpu.SemaphoreType.DMA,
            pltpu.SemaphoreType.DMA,
            pltpu.SemaphoreType.DMA,
        ],
    )(x3, emb_flat)
    return out.reshape(B, C * M, H, W)
